# Initial kernel scaffold; baseline (speedup 1.0000x reference)
import jax
import jax.numpy as jnp
from jax.experimental import pallas as pl


def kernel(ts, text, batch_idx):
    raise NotImplementedError("write your pallas kernel here")


# speedup vs baseline: 1.0067x; 1.0067x over previous
import jax
import jax.numpy as jnp
from jax.experimental import pallas as pl


def _add_kernel(ts_ref, text_ref, out_ref):
    out_ref[...] = ts_ref[...] + text_ref[...]


def kernel(ts, text, batch_idx):
    b, s, d = ts.shape
    x2 = ts.reshape(b * s, d)
    y2 = text.reshape(b * s, d)
    rows = b * s
    block_rows = 512
    grid = (rows // block_rows,)
    out = pl.pallas_call(
        _add_kernel,
        grid=grid,
        in_specs=[
            pl.BlockSpec((block_rows, d), lambda i: (i, 0)),
            pl.BlockSpec((block_rows, d), lambda i: (i, 0)),
        ],
        out_specs=pl.BlockSpec((block_rows, d), lambda i: (i, 0)),
        out_shape=jax.ShapeDtypeStruct((rows, d), ts.dtype),
    )(x2, y2)
    aux_loss = jnp.zeros((), dtype=jnp.float32)
    return (out.reshape(b, s, d), aux_loss)
